# Initial kernel scaffold; baseline (speedup 1.0000x reference)
import jax
import jax.numpy as jnp
from jax.experimental import pallas as pl


def kernel(x, edge_index, edge_attr, batch_idx, enc_W, enc_b, enc_ln_g, enc_ln_b, Wl, Wr, We, att, bias, ln_g, ln_b):
    raise NotImplementedError("write your pallas kernel here")


# speedup vs baseline: 9.7936x; 9.7936x over previous
import functools

import jax
import jax.numpy as jnp
from jax import lax
from jax.experimental import pallas as pl
from jax.experimental.pallas import tpu as pltpu
from jax.experimental.pallas import tpu_sc as plsc

N = 10000
E = 320000
D = 128
HID = 256
H = 4
C = 64
B = 64
NEG_SLOPE = 0.2

NC = 2
NS = 16
NW = NC * NS

GG = 80
NPAD = 10240
BN = 1000
BE = 1000

_f32 = jnp.float32


def _ln(h, g, b, eps=1e-5):
  mu = jnp.mean(h, axis=-1, keepdims=True)
  var = jnp.mean((h - mu) ** 2, axis=-1, keepdims=True)
  return (h - mu) / jnp.sqrt(var + eps) * g + b


def _encoder_body(x_ref, w_ref, b_ref, g_ref, bb_ref, o_ref):
  h = jnp.dot(x_ref[...], w_ref[...], preferred_element_type=_f32)
  h = h + b_ref[...]
  h = _ln(h, g_ref[...], bb_ref[...])
  o_ref[...] = jnp.maximum(h, 0.0)


def _encoder(x, enc_W, enc_b, enc_ln_g, enc_ln_b):
  return pl.pallas_call(
      _encoder_body,
      grid=(N // BN,),
      in_specs=[
          pl.BlockSpec((BN, D), lambda i: (i, 0)),
          pl.BlockSpec((D, HID), lambda i: (0, 0)),
          pl.BlockSpec((1, HID), lambda i: (0, 0)),
          pl.BlockSpec((1, HID), lambda i: (0, 0)),
          pl.BlockSpec((1, HID), lambda i: (0, 0)),
      ],
      out_specs=pl.BlockSpec((BN, HID), lambda i: (i, 0)),
      out_shape=jax.ShapeDtypeStruct((N, HID), _f32),
  )(x, enc_W, enc_b.reshape(1, HID), enc_ln_g.reshape(1, HID),
    enc_ln_b.reshape(1, HID))


def _xlxr_body(h_ref, wl_ref, wr_ref, xl_ref, xr_ref):
  h = h_ref[...]
  xl_ref[...] = jnp.dot(h, wl_ref[...], preferred_element_type=_f32)
  xr_ref[...] = jnp.dot(h, wr_ref[...], preferred_element_type=_f32)


def _xlxr(h, Wl, Wr):
  return pl.pallas_call(
      _xlxr_body,
      grid=(N // BN,),
      in_specs=[
          pl.BlockSpec((BN, HID), lambda i: (i, 0)),
          pl.BlockSpec((HID, H * C), lambda i: (0, 0)),
          pl.BlockSpec((HID, H * C), lambda i: (0, 0)),
      ],
      out_specs=[
          pl.BlockSpec((BN, H * C), lambda i: (i, 0)),
          pl.BlockSpec((BN, H * C), lambda i: (i, 0)),
      ],
      out_shape=[
          jax.ShapeDtypeStruct((N, H * C), _f32),
          jax.ShapeDtypeStruct((N, H * C), _f32),
      ],
  )(h, Wl, Wr)


def _edge_body(xls_ref, xrd_ref, ea_ref, dst_ref, we_ref, att_ref, msg_ref,
               aexp_ref):
  ee = jnp.dot(ea_ref[...], we_ref[...], preferred_element_type=_f32)
  e = xls_ref[...] + xrd_ref[...] + ee
  e = jnp.where(e > 0, e, NEG_SLOPE * e)
  ea = e * att_ref[...]
  parts = [jnp.sum(ea[:, h * C:(h + 1) * C], axis=1, keepdims=True)
           for h in range(H)]
  aexp = jnp.exp(jnp.concatenate(parts, axis=1))
  scale = jnp.concatenate(
      [jnp.broadcast_to(aexp[:, h:h + 1], (BE, C)) for h in range(H)], axis=1)
  msg_ref[...] = xls_ref[...] * scale
  lane16 = jnp.concatenate([aexp, jnp.zeros((BE, 16 - H), _f32)], axis=1)
  tile8 = jnp.concatenate([lane16] * 8, axis=1)
  slot = jnp.bitwise_and(dst_ref[...], 7)
  lane_group = lax.broadcasted_iota(jnp.int32, (BE, 128), 1) // 16
  aexp_ref[...] = jnp.where(lane_group == slot, tile8, 0.0)


def _edge_pass(xls, xrd, edge_attr, dst_col, We, att_flat):
  return pl.pallas_call(
      _edge_body,
      grid=(E // BE,),
      in_specs=[
          pl.BlockSpec((BE, H * C), lambda i: (i, 0)),
          pl.BlockSpec((BE, H * C), lambda i: (i, 0)),
          pl.BlockSpec((BE, D), lambda i: (i, 0)),
          pl.BlockSpec((BE, 1), lambda i: (i, 0)),
          pl.BlockSpec((D, H * C), lambda i: (0, 0)),
          pl.BlockSpec((1, H * C), lambda i: (0, 0)),
      ],
      out_specs=[
          pl.BlockSpec((BE, H * C), lambda i: (i, 0)),
          pl.BlockSpec((BE, 128), lambda i: (i, 0)),
      ],
      out_shape=[
          jax.ShapeDtypeStruct((E, H * C), _f32),
          jax.ShapeDtypeStruct((E, 128), _f32),
      ],
  )(xls, xrd, edge_attr, dst_col, We, att_flat)


def _finalize_body(out_ref, asum_ref, bias_ref, g_ref, b_ref, hprev_ref,
                   h_ref):
  inv = 1.0 / (asum_ref[...][:, :H] + 1e-16)
  scale = jnp.concatenate(
      [jnp.broadcast_to(inv[:, h:h + 1], (BN, C)) for h in range(H)], axis=1)
  o = out_ref[...] * scale + bias_ref[...]
  o = _ln(o, g_ref[...], b_ref[...])
  h_ref[...] = hprev_ref[...] + jnp.maximum(o, 0.0)


def _finalize(out_raw, asum, bias, ln_g, ln_b, h_prev):
  return pl.pallas_call(
      _finalize_body,
      grid=(N // BN,),
      in_specs=[
          pl.BlockSpec((BN, H * C), lambda i: (i, 0)),
          pl.BlockSpec((BN, 16), lambda i: (i, 0)),
          pl.BlockSpec((1, H * C), lambda i: (0, 0)),
          pl.BlockSpec((1, HID), lambda i: (0, 0)),
          pl.BlockSpec((1, HID), lambda i: (0, 0)),
          pl.BlockSpec((BN, HID), lambda i: (i, 0)),
      ],
      out_specs=pl.BlockSpec((BN, HID), lambda i: (i, 0)),
      out_shape=jax.ShapeDtypeStruct((N, HID), _f32),
  )(out_raw, asum, bias.reshape(1, H * C), ln_g.reshape(1, HID),
    ln_b.reshape(1, HID), h_prev)


def _pool_body(h_ref, bi_ref, o_ref, sums, cnts):
  i = pl.program_id(0)

  @pl.when(i == 0)
  def _():
    sums[...] = jnp.zeros_like(sums)
    cnts[...] = jnp.zeros_like(cnts)

  oh = (bi_ref[...] == lax.broadcasted_iota(jnp.int32, (BN, B), 1))
  oh = oh.astype(_f32)
  sums[...] += lax.dot_general(oh, h_ref[...], (((0,), (0,)), ((), ())),
                               preferred_element_type=_f32)
  cnts[...] += jnp.sum(oh, axis=0, keepdims=True)

  @pl.when(i == N // BN - 1)
  def _():
    o_ref[...] = sums[...] / jnp.maximum(cnts[...], 1.0).reshape(B, 1)


def _pool(h, batch_idx):
  return pl.pallas_call(
      _pool_body,
      grid=(N // BN,),
      in_specs=[
          pl.BlockSpec((BN, HID), lambda i: (i, 0)),
          pl.BlockSpec((BN, 1), lambda i: (i, 0)),
      ],
      out_specs=pl.BlockSpec((B, HID), lambda i: (0, 0)),
      out_shape=jax.ShapeDtypeStruct((B, HID), _f32),
      scratch_shapes=[
          pltpu.VMEM((B, HID), _f32),
          pltpu.VMEM((1, B), _f32),
      ],
  )(h, batch_idx.reshape(N, 1))


_EPW = E // NW
_EPT = E // NS
_RPT = NPAD // NS


def _gather_body(xl_hbm, xr_hbm, src_hbm, dst_hbm, xls_hbm, xrd_hbm,
                 idx_s, idx_d, rows_a, rows_b, sem_a, sem_b):
  wid = lax.axis_index("s") * NC + lax.axis_index("c")

  def body(i, carry):
    base = wid * _EPW + i * GG
    pltpu.sync_copy(src_hbm.at[pl.ds(base, GG)], idx_s)
    pltpu.sync_copy(dst_hbm.at[pl.ds(base, GG)], idx_d)
    a = pltpu.async_copy(xl_hbm.at[idx_s], rows_a, sem_a)
    b = pltpu.async_copy(xr_hbm.at[idx_d], rows_b, sem_b)
    a.wait()
    b.wait()
    pltpu.sync_copy(rows_a, xls_hbm.at[pl.ds(base, GG)])
    pltpu.sync_copy(rows_b, xrd_hbm.at[pl.ds(base, GG)])
    return carry

  lax.fori_loop(0, _EPW // GG, body, 0, unroll=False)


@functools.cache
def _sc_gather_kernel():
  return pl.kernel(
      _gather_body,
      out_type=(
          jax.ShapeDtypeStruct((E, H * C), _f32),
          jax.ShapeDtypeStruct((E, H * C), _f32),
      ),
      mesh=plsc.VectorSubcoreMesh(
          core_axis_name="c", subcore_axis_name="s",
          num_cores=NC, num_subcores=NS),
      scratch_types=[
          pltpu.VMEM((GG,), jnp.int32),
          pltpu.VMEM((GG,), jnp.int32),
          pltpu.VMEM((GG, H * C), _f32),
          pltpu.VMEM((GG, H * C), _f32),
          pltpu.SemaphoreType.DMA,
          pltpu.SemaphoreType.DMA,
      ],
  )


def _sc_gather(*args):
  return _sc_gather_kernel()(*args)


_APK = NPAD // 8
_APT = _APK // NS


def _scatter_body(msg_hbm, aexp_hbm, dst_hbm, z_hbm, out_hbm, asum_hbm,
                  idx_v, idx2_v, rows_v, aexp_v, acc, asum_acc, sem):
  cid = lax.axis_index("c")
  tid = lax.axis_index("s")
  r0 = tid * _RPT

  def zinit(j, carry):
    rr = r0 + j * GG
    pltpu.sync_copy(z_hbm.at[pl.ds(rr, GG)], rows_v)
    pltpu.sync_copy(rows_v, acc.at[pl.ds(rr, GG)])
    return carry

  lax.fori_loop(0, _RPT // GG, zinit, 0, unroll=False)
  pltpu.sync_copy(z_hbm.at[pl.ds(tid * _APT, _APT)], aexp_v)
  pltpu.sync_copy(aexp_v, asum_acc.at[pl.ds(tid * _APT, _APT)])
  plsc.subcore_barrier()

  def body(i, carry):
    base = tid * _EPT + i * GG
    pltpu.sync_copy(dst_hbm.at[pl.ds(base, GG)], idx_v)
    pltpu.sync_copy(msg_hbm.at[pl.ds(base, GG), pl.ds(cid * 128, 128)],
                    rows_v)
    pltpu.sync_copy(rows_v, acc.at[idx_v], add=True)

    @pl.when(cid == 0)
    def _():
      pltpu.sync_copy(aexp_hbm.at[pl.ds(base, GG)], aexp_v)
      for k in range(GG // 16):
        sl = pl.ds(k * 16, 16)
        idx2_v[sl] = jax.lax.shift_right_logical(idx_v[sl], 3)
      pltpu.sync_copy(aexp_v, asum_acc.at[idx2_v], add=True)

    return carry

  lax.fori_loop(0, _EPT // GG, body, 0, unroll=False)
  plsc.subcore_barrier()

  def flush(j, carry):
    rr = r0 + j * GG
    pltpu.sync_copy(acc.at[pl.ds(rr, GG)], rows_v)
    pltpu.sync_copy(rows_v, out_hbm.at[pl.ds(rr, GG), pl.ds(cid * 128, 128)])
    return carry

  lax.fori_loop(0, _RPT // GG, flush, 0, unroll=False)

  @pl.when(cid == 0)
  def _():
    pltpu.sync_copy(asum_acc.at[pl.ds(tid * _APT, _APT)], aexp_v)
    pltpu.sync_copy(aexp_v, asum_hbm.at[pl.ds(tid * _APT, _APT)])


@functools.cache
def _sc_scatter_kernel():
  return pl.kernel(
      _scatter_body,
      out_type=(
          jax.ShapeDtypeStruct((NPAD, H * C), _f32),
          jax.ShapeDtypeStruct((_APK, 128), _f32),
      ),
      mesh=plsc.VectorSubcoreMesh(
          core_axis_name="c", subcore_axis_name="s",
          num_cores=NC, num_subcores=NS),
      scratch_types=[
          pltpu.VMEM((GG,), jnp.int32),
          pltpu.VMEM((GG,), jnp.int32),
          pltpu.VMEM((GG, 128), _f32),
          pltpu.VMEM((GG, 128), _f32),
          pltpu.VMEM_SHARED((NPAD, 128), _f32),
          pltpu.VMEM_SHARED((_APK, 128), _f32),
          pltpu.SemaphoreType.DMA,
      ],
  )


def _sc_scatter(*args):
  return _sc_scatter_kernel()(*args)


def kernel(x, edge_index, edge_attr, batch_idx, enc_W, enc_b, enc_ln_g,
           enc_ln_b, Wl, Wr, We, att, bias, ln_g, ln_b):
  src = edge_index[0]
  dst = edge_index[1]
  dst_col = dst.reshape(E, 1)
  zeros_n = jnp.zeros((NPAD, 128), _f32)

  h = _encoder(x, enc_W, enc_b, enc_ln_g, enc_ln_b)
  for l in range(3):
    xl, xr = _xlxr(h, Wl[l], Wr[l])
    xls, xrd = _sc_gather(xl, xr, src, dst)
    msg, aexp = _edge_pass(xls, xrd, edge_attr, dst_col, We[l],
                           att[l].reshape(1, H * C))
    out_raw, asum_pk = _sc_scatter(msg, aexp, dst, zeros_n)
    asum = asum_pk.reshape(NPAD, 16)[:N]
    h = _finalize(out_raw[:N], asum, bias[l], ln_g[l], ln_b[l], h)
  return _pool(h, batch_idx)
